# Initial kernel scaffold; baseline (speedup 1.0000x reference)
import jax
import jax.numpy as jnp
from jax.experimental import pallas as pl


def kernel(idx, idx1, idx2, idx3, cos_emb, conv1_w, conv1_b, conv2_w, conv2_b, out_w, out_b):
    raise NotImplementedError("write your pallas kernel here")


# speedup vs baseline: 1.5449x; 1.5449x over previous
import jax
import jax.numpy as jnp
from jax.experimental import pallas as pl
from jax.experimental.pallas import tpu as pltpu

_BS = 16


def _fused_body(idx_ref, emb_ref, w1_ref, b1_ref, w2_ref, b2_ref, w3_ref,
                b3_ref, out_ref):
    x = idx_ref[...]
    nrm = jnp.maximum(jnp.sqrt(jnp.sum(x * x, axis=-1, keepdims=True)), 1e-12)
    f = x / nrm
    C = jnp.einsum("bnd,bmd->bnm", f, f)[:, :20, :20]
    base = (jnp.floor(C * 10.0) + 1.0).astype(jnp.int32)
    bins = jnp.where(C >= 1.0, 11, jnp.where(C < 0.1, 1, base))
    emb = emb_ref[...]
    bs = bins.shape[0]
    e1 = jnp.zeros((bs, 20, 16, 20), jnp.float32)
    for k in range(1, 12):
        mask = (bins == k).astype(jnp.float32)
        e1 = e1 + mask[:, :, None, :] * emb[k][None, None, :, None]
    xp1 = jnp.pad(e1, ((0, 0), (2, 2), (2, 2), (0, 0)))
    cols = [xp1[:, dh:dh + 22, dw:dw + 18, :]
            for dh in range(3) for dw in range(3)]
    x1 = jnp.concatenate(cols, axis=-1)
    y1 = jax.lax.dot_general(
        x1.reshape(bs * 22 * 18, 180), w1_ref[...],
        dimension_numbers=(((1,), (0,)), ((), ())),
        preferred_element_type=jnp.float32)
    y1 = jax.nn.relu(y1 + b1_ref[...]).reshape(bs, 22, 18, 32)
    a = y1.reshape(bs, 11, 2, 18, 32)
    hmax = jnp.maximum(a[:, :, 0], a[:, :, 1])
    b = hmax.reshape(bs, 11, 9, 2, 32)
    p1 = jnp.maximum(b[:, :, :, 0], b[:, :, :, 1])
    xp2 = jnp.pad(p1, ((0, 0), (2, 2), (2, 2), (0, 0)))
    cols2 = [xp2[:, dh:dh + 13, dw:dw + 11, :]
             for dh in range(3) for dw in range(3)]
    x2 = jnp.concatenate(cols2, axis=-1)
    y2 = jax.lax.dot_general(
        x2.reshape(bs * 13 * 11, 288), w2_ref[...],
        dimension_numbers=(((1,), (0,)), ((), ())),
        preferred_element_type=jnp.float32)
    y2 = jax.nn.relu(y2 + b2_ref[...]).reshape(bs, 13, 11, 64)
    c = y2[:, 0:12].reshape(bs, 6, 2, 11, 64)
    hmax2 = jnp.maximum(c[:, :, 0], c[:, :, 1])
    e2 = hmax2[:, :, 0:10].reshape(bs, 6, 5, 2, 64)
    p2 = jnp.maximum(e2[:, :, :, 0], e2[:, :, :, 1])
    flat = p2.reshape(bs, 1920)
    out_ref[...] = jax.lax.dot_general(
        flat, w3_ref[...], dimension_numbers=(((1,), (0,)), ((), ())),
        preferred_element_type=jnp.float32) + b3_ref[...]


def kernel(idx, idx1, idx2, idx3, cos_emb, conv1_w, conv1_b, conv2_w,
           conv2_b, out_w, out_b):
    B = idx.shape[0]
    bs = _BS
    w1 = conv1_w.transpose(2, 3, 1, 0).reshape(180, 32)
    w2 = conv2_w.transpose(2, 3, 1, 0).reshape(288, 64)
    w3 = out_w.reshape(3, 64, 6, 5).transpose(2, 3, 1, 0).reshape(1920, 3)
    b1 = conv1_b.reshape(1, 32)
    b2 = conv2_b.reshape(1, 64)
    b3 = out_b.reshape(1, 3)

    grid = (B // bs,)
    fixed = lambda i: (0, 0)
    out = pl.pallas_call(
        _fused_body,
        grid=grid,
        in_specs=[
            pl.BlockSpec((bs, 21, 768), lambda i: (i, 0, 0)),
            pl.BlockSpec((12, 16), fixed),
            pl.BlockSpec((180, 32), fixed),
            pl.BlockSpec((1, 32), fixed),
            pl.BlockSpec((288, 64), fixed),
            pl.BlockSpec((1, 64), fixed),
            pl.BlockSpec((1920, 3), fixed),
            pl.BlockSpec((1, 3), fixed),
        ],
        out_specs=pl.BlockSpec((bs, 3), lambda i: (i, 0)),
        out_shape=jax.ShapeDtypeStruct((B, 3), jnp.float32),
        compiler_params=pltpu.CompilerParams(
            dimension_semantics=("parallel",)),
    )(idx, cos_emb, w1, b1, w2, b2, w3, b3)
    return out
